# baseline (speedup 1.0000x reference)
import jax
import jax.numpy as jnp
from jax.experimental import pallas as pl
from jax.experimental.pallas import tpu as pltpu

_SEQ = 1024
_DIM = 64
_BI = 16
_WIN = _SEQ + 128


def _body(table_ref, x_ref, o_ref):
    i0 = pl.program_id(0) * _BI
    for r in range(_BI):
        s = _SEQ - 1 - (i0 + r)
        a = pl.multiple_of((s // 128) * 128, 128)
        b = s - a
        coarse = table_ref[:, pl.ds(a, _WIN)]
        win = pltpu.roll(coarse, (_WIN - b) % _WIN, axis=1)
        o_ref[0, r] = x_ref[0, r] + win[:, :_SEQ]


def kernel(x, relative_embedding):
    rt_t = relative_embedding[::-1].T
    rt_p = jnp.pad(rt_t, ((0, 0), (0, 1)))
    xt = jnp.transpose(x, (0, 1, 3, 2))
    out = pl.pallas_call(
        _body,
        grid=(_SEQ // _BI,),
        in_specs=[
            pl.BlockSpec((_DIM, 2 * _SEQ), lambda i: (0, 0)),
            pl.BlockSpec((1, _BI, _DIM, _SEQ), lambda i: (0, i, 0, 0)),
        ],
        out_specs=pl.BlockSpec((1, _BI, _DIM, _SEQ), lambda i: (0, i, 0, 0)),
        out_shape=jax.ShapeDtypeStruct(xt.shape, x.dtype),
    )(rt_p, xt)
    return jnp.transpose(out, (0, 1, 3, 2))


# speedup vs baseline: 1.0374x; 1.0374x over previous
import jax
import jax.numpy as jnp
from jax.experimental import pallas as pl
from jax.experimental.pallas import tpu as pltpu

_SEQ = 1024
_DIM = 64
_BI = 32
_WIN = _SEQ + 128


def _body(table_ref, x_ref, o_ref):
    i0 = pl.program_id(0) * _BI
    for r in range(_BI):
        s = _SEQ - 1 - (i0 + r)
        a = pl.multiple_of((s // 128) * 128, 128)
        b = s - a
        coarse = table_ref[:, pl.ds(a, _WIN)]
        win = pltpu.roll(coarse, (_WIN - b) % _WIN, axis=1)
        o_ref[0, r] = x_ref[0, r] + win[:, :_SEQ]


def kernel(x, relative_embedding):
    rt_t = relative_embedding[::-1].T
    rt_p = jnp.pad(rt_t, ((0, 0), (0, 1)))
    xt = jnp.transpose(x, (0, 1, 3, 2))
    out = pl.pallas_call(
        _body,
        grid=(_SEQ // _BI,),
        in_specs=[
            pl.BlockSpec((_DIM, 2 * _SEQ), lambda i: (0, 0)),
            pl.BlockSpec((1, _BI, _DIM, _SEQ), lambda i: (0, i, 0, 0)),
        ],
        out_specs=pl.BlockSpec((1, _BI, _DIM, _SEQ), lambda i: (0, i, 0, 0)),
        out_shape=jax.ShapeDtypeStruct(xt.shape, x.dtype),
    )(rt_p, xt)
    return jnp.transpose(out, (0, 1, 3, 2))
